# baseline (speedup 1.0000x reference)
import jax
import jax.numpy as jnp
from jax import lax
from jax.experimental import pallas as pl
from jax.experimental.pallas import tpu as pltpu
from jax.experimental.pallas import tpu_sc as plsc

H = 512
W = 512
HW = H * W
NPT = 131072
NB = 8
NSUB = 16
SLAB = HW // NSUB
SLAB_SHIFT = 14
CH = 2048
NCHUNK = NPT // CH
L = 16


def _any(mask):
    return plsc.all_reduce_population_count(mask)[0] > 0


def _pcnt(mask):
    return plsc.all_reduce_population_count(mask)[0]


def _body(pts_ref, col_ref, depth_hbm, img_hbm, index_hbm,
          depth_ref, idbuf_ref, img0_ref, img1_ref, img2_ref,
          sbufs_a, sbufs_b, lbuf, zbuf, ibuf, r0buf, r1buf, r2buf,
          sem_a, sem_b):
    cax = lax.axis_index("c")
    o = lax.axis_index("s")
    iota = lax.broadcasted_iota(jnp.int32, (L,), 0)

    def batch_body(t, _bc):
        b = cax * 4 + t
        pbase = b * 4 * NPT
        cbase = b * 3 * NPT

        def init_body(i, _):
            dsl = pl.ds(i * L, L)
            depth_ref[dsl] = jnp.full((L,), jnp.inf, jnp.float32)
            idbuf_ref[dsl] = jnp.full((L,), NPT, jnp.int32)
            img0_ref[dsl] = jnp.zeros((L,), jnp.float32)
            img1_ref[dsl] = jnp.zeros((L,), jnp.float32)
            img2_ref[dsl] = jnp.zeros((L,), jnp.float32)
            return 0

        lax.fori_loop(0, SLAB // L, init_body, 0)

        def issue(ci, bufs, sem, mode):
            off = ci * CH
            for r in range(3):
                pltpu.async_copy(
                    pts_ref.at[pl.ds(pbase + r * NPT + off, CH)],
                    bufs[r], sem)
            if mode == 2:
                for r in range(3):
                    pltpu.async_copy(
                        col_ref.at[pl.ds(cbase + r * NPT + off, CH)],
                        bufs[3 + r], sem)

        def drain(bufs, sem, mode):
            n = 3 if mode == 1 else 6
            for r in range(n):
                pltpu.make_async_copy(
                    pts_ref.at[pl.ds(0, CH)], bufs[r], sem).wait()

        def process_chunk(ci, bufs, mode):
            off = ci * CH
            x_ref, y_ref, z_ref = bufs[0], bufs[1], bufs[2]

            NREG = 4
            REG = CH // NREG

            def ph_a(j4, cnts):
                new = []
                for k in range(NREG):
                    cnt = cnts[k]
                    j = j4 * NREG + k
                    dsl = pl.ds(j * L, L)
                    xv = x_ref[dsl]
                    yv = y_ref[dsl]
                    zv = z_ref[dsl]
                    u = jnp.minimum((xv * W).astype(jnp.int32), W - 1)
                    v = jnp.minimum((yv * H).astype(jnp.int32), H - 1)
                    pix = lax.shift_left(v, 9) + u
                    mine = lax.shift_right_logical(pix, SLAB_SHIFT) == o
                    loc = pix & (SLAB - 1)
                    dst = pl.ds(k * REG + cnt, L)
                    plsc.store_compressed(lbuf.at[dst], loc, mask=mine)
                    plsc.store_compressed(zbuf.at[dst], zv, mask=mine)
                    if mode == 2:
                        idv = (off + j * L) + iota
                        plsc.store_compressed(ibuf.at[dst], idv, mask=mine)
                        plsc.store_compressed(r0buf.at[dst], bufs[3][dsl],
                                              mask=mine)
                        plsc.store_compressed(r1buf.at[dst], bufs[4][dsl],
                                              mask=mine)
                        plsc.store_compressed(r2buf.at[dst], bufs[5][dsl],
                                              mask=mine)
                    new.append(cnt + _pcnt(mine))
                return tuple(new)

            cnts = lax.fori_loop(0, CH // L // NREG, ph_a,
                                 tuple(jnp.int32(0) for _ in range(NREG)))

            def make_ph_b(base, cnt):
                if mode == 1:
                    def ph_b(k, _):
                        dsl = pl.ds(base + k * L, L)
                        locv = lbuf[dsl] & (SLAB - 1)
                        zv = zbuf[dsl]
                        valid = (k * L + iota) < cnt
                        cur = plsc.load_gather(depth_ref, [locv])
                        w0 = valid & (zv < cur)

                        def body(m):
                            plsc.store_scatter(depth_ref, [locv], zv, mask=m)
                            cur2 = plsc.load_gather(depth_ref, [locv])
                            return m & (zv < cur2)

                        lax.while_loop(_any, body, w0)
                        return 0
                else:
                    def ph_b(k, _):
                        dsl = pl.ds(base + k * L, L)
                        locv = lbuf[dsl] & (SLAB - 1)
                        zv = zbuf[dsl]
                        valid = (k * L + iota) < cnt
                        curz = plsc.load_gather(depth_ref, [locv])
                        m0 = valid & (zv == curz)

                        @pl.when(_any(m0))
                        def _():
                            idv = ibuf[dsl]
                            c0v = r0buf[dsl]
                            c1v = r1buf[dsl]
                            c2v = r2buf[dsl]
                            curi = plsc.load_gather(idbuf_ref, [locv])
                            w0 = m0 & (idv < curi)

                            def body(m):
                                plsc.store_scatter(idbuf_ref, [locv], idv,
                                                   mask=m)
                                plsc.store_scatter(img0_ref, [locv], c0v,
                                                   mask=m)
                                plsc.store_scatter(img1_ref, [locv], c1v,
                                                   mask=m)
                                plsc.store_scatter(img2_ref, [locv], c2v,
                                                   mask=m)
                                ri = plsc.load_gather(idbuf_ref, [locv])
                                g0 = plsc.load_gather(img0_ref, [locv])
                                g1 = plsc.load_gather(img1_ref, [locv])
                                g2 = plsc.load_gather(img2_ref, [locv])
                                torn = ((c0v != g0) | (c1v != g1)
                                        | (c2v != g2))
                                return m0 & ((idv < ri)
                                             | ((idv == ri) & torn))

                            lax.while_loop(_any, body, w0)
                        return 0
                return ph_b

            for r in range(NREG):
                cnt_r = cnts[r]
                nb_r = (cnt_r + (L - 1)) // L
                lax.fori_loop(0, nb_r, make_ph_b(r * REG, cnt_r), 0)

        for mode in (1, 2):
            issue(0, sbufs_a, sem_a, mode)
            issue(1, sbufs_b, sem_b, mode)

            def q_body(q, _, mode=mode):
                for par, bufs, sem in ((0, sbufs_a, sem_a),
                                       (1, sbufs_b, sem_b)):
                    ci = 2 * q + par
                    drain(bufs, sem, mode)
                    process_chunk(ci, bufs, mode)

                    @pl.when(ci + 2 < NCHUNK)
                    def _():
                        issue(ci + 2, bufs, sem, mode)
                return 0

            lax.fori_loop(0, NCHUNK // 2, q_body, 0)

        def fin_body(i, _):
            dsl = pl.ds(i * L, L)
            idv = idbuf_ref[dsl]
            dv = depth_ref[dsl]
            valid = idv < NPT
            depth_ref[dsl] = jnp.where(valid, dv, jnp.float32(0.0))
            idbuf_ref[dsl] = jnp.where(valid, idv, jnp.int32(-1))
            return 0

        lax.fori_loop(0, SLAB // L, fin_body, 0)
        gbase = b * HW + o * SLAB
        pltpu.sync_copy(depth_ref, depth_hbm.at[pl.ds(gbase, SLAB)])
        pltpu.sync_copy(idbuf_ref, index_hbm.at[pl.ds(gbase, SLAB)])
        for ch, cref in ((0, img0_ref), (1, img1_ref), (2, img2_ref)):
            ibase = (b * 3 + ch) * HW + o * SLAB
            pltpu.sync_copy(cref, img_hbm.at[pl.ds(ibase, SLAB)])
        return 0

    lax.fori_loop(0, 4, batch_body, 0)


_proj = pl.kernel(
    _body,
    out_type=(
        jax.ShapeDtypeStruct((NB * HW,), jnp.float32),
        jax.ShapeDtypeStruct((NB * 3 * HW,), jnp.float32),
        jax.ShapeDtypeStruct((NB * HW,), jnp.int32),
    ),
    mesh=plsc.VectorSubcoreMesh(core_axis_name="c", subcore_axis_name="s"),
    scratch_types=[
        pltpu.VMEM((SLAB,), jnp.float32),
        pltpu.VMEM((SLAB,), jnp.int32),
        pltpu.VMEM((SLAB,), jnp.float32),
        pltpu.VMEM((SLAB,), jnp.float32),
        pltpu.VMEM((SLAB,), jnp.float32),
        [pltpu.VMEM((CH,), jnp.float32) for _ in range(6)],
        [pltpu.VMEM((CH,), jnp.float32) for _ in range(6)],
        pltpu.VMEM((CH,), jnp.int32),
        pltpu.VMEM((CH,), jnp.float32),
        pltpu.VMEM((CH,), jnp.int32),
        pltpu.VMEM((CH,), jnp.float32),
        pltpu.VMEM((CH,), jnp.float32),
        pltpu.VMEM((CH,), jnp.float32),
        pltpu.SemaphoreType.DMA,
        pltpu.SemaphoreType.DMA,
    ],
    compiler_params=pltpu.CompilerParams(needs_layout_passes=False),
)


def kernel(points, colors):
    B, _, N = points.shape
    depth, img, index = _proj(points.reshape(-1), colors.reshape(-1))
    return (depth.reshape(B, H, W),
            img.reshape(B, 3, H, W),
            index.reshape(B, H, W))


# speedup vs baseline: 1.2108x; 1.2108x over previous
import jax
import jax.numpy as jnp
from jax import lax
from jax.experimental import pallas as pl
from jax.experimental.pallas import tpu as pltpu
from jax.experimental.pallas import tpu_sc as plsc

H = 512
W = 512
HW = H * W
NPT = 131072
NB = 8
NSUB = 16
SLAB = HW // NSUB
SLAB_SHIFT = 14
CH = 2048
NCHUNK = NPT // CH
L = 16


def _any(mask):
    return plsc.all_reduce_population_count(mask)[0] > 0


def _pcnt(mask):
    return plsc.all_reduce_population_count(mask)[0]


def _body(pts_ref, col_ref, depth_hbm, img_hbm, index_hbm,
          depth_ref, idbuf_ref, img0_ref, img1_ref, img2_ref,
          sbufs_a, sbufs_b, lbuf,
          sem_a, sem_b):
    cax = lax.axis_index("c")
    o = lax.axis_index("s")
    iota = lax.broadcasted_iota(jnp.int32, (L,), 0)

    def batch_body(t, _bc):
        b = cax * 4 + t
        pbase = b * 4 * NPT
        cbase = b * 3 * NPT

        def init_body(i, _):
            dsl = pl.ds(i * L, L)
            depth_ref[dsl] = jnp.full((L,), jnp.inf, jnp.float32)
            idbuf_ref[dsl] = jnp.full((L,), NPT, jnp.int32)
            img0_ref[dsl] = jnp.zeros((L,), jnp.float32)
            img1_ref[dsl] = jnp.zeros((L,), jnp.float32)
            img2_ref[dsl] = jnp.zeros((L,), jnp.float32)
            return 0

        lax.fori_loop(0, SLAB // L, init_body, 0)

        def issue(ci, bufs, sem, mode):
            off = ci * CH
            for r in range(3):
                pltpu.async_copy(
                    pts_ref.at[pl.ds(pbase + r * NPT + off, CH)],
                    bufs[r], sem)
            if mode == 2:
                for r in range(3):
                    pltpu.async_copy(
                        col_ref.at[pl.ds(cbase + r * NPT + off, CH)],
                        bufs[3 + r], sem)

        def drain(bufs, sem, mode):
            n = 3 if mode == 1 else 6
            for r in range(n):
                pltpu.make_async_copy(
                    pts_ref.at[pl.ds(0, CH)], bufs[r], sem).wait()

        def process_chunk(ci, bufs, mode):
            off = ci * CH
            x_ref, y_ref = bufs[0], bufs[1]

            NREG = 4
            REG = CH // NREG

            def ph_a(j4, cnts):
                new = []
                for k in range(NREG):
                    cnt = cnts[k]
                    j = j4 * NREG + k
                    dsl = pl.ds(j * L, L)
                    xv = x_ref[dsl]
                    yv = y_ref[dsl]
                    u = jnp.minimum((xv * W).astype(jnp.int32), W - 1)
                    v = jnp.minimum((yv * H).astype(jnp.int32), H - 1)
                    pix = lax.shift_left(v, 9) + u
                    mine = lax.shift_right_logical(pix, SLAB_SHIFT) == o
                    packed = (pix & (SLAB - 1)) | lax.shift_left(
                        j * L + iota, SLAB_SHIFT)
                    plsc.store_compressed(lbuf.at[pl.ds(k * REG + cnt, L)],
                                          packed, mask=mine)
                    new.append(cnt + _pcnt(mine))
                return tuple(new)

            cnts = lax.fori_loop(0, CH // L // NREG, ph_a,
                                 tuple(jnp.int32(0) for _ in range(NREG)))

            def make_ph_b(base, cnt):
                if mode == 1:
                    def ph_b(k, _):
                        dsl = pl.ds(base + k * L, L)
                        pk = lbuf[dsl]
                        locv = pk & (SLAB - 1)
                        posv = lax.shift_right_logical(pk, SLAB_SHIFT) \
                            & (CH - 1)
                        zv = plsc.load_gather(bufs[2], [posv])
                        valid = (k * L + iota) < cnt
                        cur = plsc.load_gather(depth_ref, [locv])
                        w0 = valid & (zv < cur)

                        def body(m):
                            plsc.store_scatter(depth_ref, [locv], zv, mask=m)
                            cur2 = plsc.load_gather(depth_ref, [locv])
                            return m & (zv < cur2)

                        lax.while_loop(_any, body, w0)
                        return 0
                else:
                    def ph_b(k, _):
                        dsl = pl.ds(base + k * L, L)
                        pk = lbuf[dsl]
                        locv = pk & (SLAB - 1)
                        posv = lax.shift_right_logical(pk, SLAB_SHIFT) \
                            & (CH - 1)
                        zv = plsc.load_gather(bufs[2], [posv])
                        valid = (k * L + iota) < cnt
                        curz = plsc.load_gather(depth_ref, [locv])
                        m0 = valid & (zv == curz)

                        @pl.when(_any(m0))
                        def _():
                            idv = off + posv
                            c0v = plsc.load_gather(bufs[3], [posv])
                            c1v = plsc.load_gather(bufs[4], [posv])
                            c2v = plsc.load_gather(bufs[5], [posv])
                            curi = plsc.load_gather(idbuf_ref, [locv])
                            w0 = m0 & (idv < curi)

                            def body(m):
                                plsc.store_scatter(idbuf_ref, [locv], idv,
                                                   mask=m)
                                plsc.store_scatter(img0_ref, [locv], c0v,
                                                   mask=m)
                                plsc.store_scatter(img1_ref, [locv], c1v,
                                                   mask=m)
                                plsc.store_scatter(img2_ref, [locv], c2v,
                                                   mask=m)
                                ri = plsc.load_gather(idbuf_ref, [locv])
                                g0 = plsc.load_gather(img0_ref, [locv])
                                g1 = plsc.load_gather(img1_ref, [locv])
                                g2 = plsc.load_gather(img2_ref, [locv])
                                torn = ((c0v != g0) | (c1v != g1)
                                        | (c2v != g2))
                                return m0 & ((idv < ri)
                                             | ((idv == ri) & torn))

                            lax.while_loop(_any, body, w0)
                        return 0
                return ph_b

            for r in range(NREG):
                cnt_r = cnts[r]
                nb_r = (cnt_r + (L - 1)) // L
                lax.fori_loop(0, nb_r, make_ph_b(r * REG, cnt_r), 0)

        for mode in (1, 2):
            issue(0, sbufs_a, sem_a, mode)
            issue(1, sbufs_b, sem_b, mode)

            def q_body(q, _, mode=mode):
                for par, bufs, sem in ((0, sbufs_a, sem_a),
                                       (1, sbufs_b, sem_b)):
                    ci = 2 * q + par
                    drain(bufs, sem, mode)
                    process_chunk(ci, bufs, mode)

                    @pl.when(ci + 2 < NCHUNK)
                    def _():
                        issue(ci + 2, bufs, sem, mode)
                return 0

            lax.fori_loop(0, NCHUNK // 2, q_body, 0)

        def fin_body(i, _):
            dsl = pl.ds(i * L, L)
            idv = idbuf_ref[dsl]
            dv = depth_ref[dsl]
            valid = idv < NPT
            depth_ref[dsl] = jnp.where(valid, dv, jnp.float32(0.0))
            idbuf_ref[dsl] = jnp.where(valid, idv, jnp.int32(-1))
            return 0

        lax.fori_loop(0, SLAB // L, fin_body, 0)
        gbase = b * HW + o * SLAB
        pltpu.sync_copy(depth_ref, depth_hbm.at[pl.ds(gbase, SLAB)])
        pltpu.sync_copy(idbuf_ref, index_hbm.at[pl.ds(gbase, SLAB)])
        for ch, cref in ((0, img0_ref), (1, img1_ref), (2, img2_ref)):
            ibase = (b * 3 + ch) * HW + o * SLAB
            pltpu.sync_copy(cref, img_hbm.at[pl.ds(ibase, SLAB)])
        return 0

    lax.fori_loop(0, 4, batch_body, 0)


_proj = pl.kernel(
    _body,
    out_type=(
        jax.ShapeDtypeStruct((NB * HW,), jnp.float32),
        jax.ShapeDtypeStruct((NB * 3 * HW,), jnp.float32),
        jax.ShapeDtypeStruct((NB * HW,), jnp.int32),
    ),
    mesh=plsc.VectorSubcoreMesh(core_axis_name="c", subcore_axis_name="s"),
    scratch_types=[
        pltpu.VMEM((SLAB,), jnp.float32),
        pltpu.VMEM((SLAB,), jnp.int32),
        pltpu.VMEM((SLAB,), jnp.float32),
        pltpu.VMEM((SLAB,), jnp.float32),
        pltpu.VMEM((SLAB,), jnp.float32),
        [pltpu.VMEM((CH,), jnp.float32) for _ in range(6)],
        [pltpu.VMEM((CH,), jnp.float32) for _ in range(6)],
        pltpu.VMEM((CH,), jnp.int32),
        pltpu.SemaphoreType.DMA,
        pltpu.SemaphoreType.DMA,
    ],
    compiler_params=pltpu.CompilerParams(needs_layout_passes=False),
)


def kernel(points, colors):
    B, _, N = points.shape
    depth, img, index = _proj(points.reshape(-1), colors.reshape(-1))
    return (depth.reshape(B, H, W),
            img.reshape(B, 3, H, W),
            index.reshape(B, H, W))


# speedup vs baseline: 1.6995x; 1.4036x over previous
import jax
import jax.numpy as jnp
from jax import lax
from jax.experimental import pallas as pl
from jax.experimental.pallas import tpu as pltpu
from jax.experimental.pallas import tpu_sc as plsc

H = 512
W = 512
HW = H * W
NPT = 131072
NB = 8
NSUB = 16
SLAB = HW // NSUB
SLAB_SHIFT = 14
CH = 2048
NCHUNK = NPT // CH
L = 16
NREG = 4
REG = CH // NREG
CAP = 20480


def _any(mask):
    return plsc.all_reduce_population_count(mask)[0] > 0


def _pcnt(mask):
    return plsc.all_reduce_population_count(mask)[0]


def _body(pts_ref, col_ref, depth_hbm, img_hbm, index_hbm,
          depth_ref, idbuf_ref, img0_ref, img1_ref, img2_ref,
          sbufs_a, sbufs_b, lbuf, persist_ref, segs_ref, lens_ref,
          sem_a, sem_b):
    cax = lax.axis_index("c")
    o = lax.axis_index("s")
    iota = lax.broadcasted_iota(jnp.int32, (L,), 0)

    def batch_body(t, _bc):
        b = cax * 4 + t
        pbase = b * 4 * NPT
        cbase = b * 3 * NPT

        def init_body(i, _):
            dsl = pl.ds(i * L, L)
            depth_ref[dsl] = jnp.full((L,), jnp.inf, jnp.float32)
            idbuf_ref[dsl] = jnp.full((L,), NPT, jnp.int32)
            img0_ref[dsl] = jnp.zeros((L,), jnp.float32)
            img1_ref[dsl] = jnp.zeros((L,), jnp.float32)
            img2_ref[dsl] = jnp.zeros((L,), jnp.float32)
            return 0

        lax.fori_loop(0, SLAB // L, init_body, 0)

        def issue(ci, bufs, sem, mode):
            off = ci * CH
            for r in range(3):
                pltpu.async_copy(
                    pts_ref.at[pl.ds(pbase + r * NPT + off, CH)],
                    bufs[r], sem)
            if mode == 2:
                for r in range(3):
                    pltpu.async_copy(
                        col_ref.at[pl.ds(cbase + r * NPT + off, CH)],
                        bufs[3 + r], sem)

        def drain(bufs, sem, mode):
            n = 3 if mode == 1 else 6
            for r in range(n):
                pltpu.make_async_copy(
                    pts_ref.at[pl.ds(0, CH)], bufs[r], sem).wait()

        def ph_a(bufs):
            x_ref, y_ref = bufs[0], bufs[1]

            def step(j4, cnts):
                new = []
                for k in range(NREG):
                    cnt = cnts[k]
                    j = j4 * NREG + k
                    dsl = pl.ds(j * L, L)
                    xv = x_ref[dsl]
                    yv = y_ref[dsl]
                    u = jnp.minimum((xv * W).astype(jnp.int32), W - 1)
                    v = jnp.minimum((yv * H).astype(jnp.int32), H - 1)
                    pix = lax.shift_left(v, 9) + u
                    mine = lax.shift_right_logical(pix, SLAB_SHIFT) == o
                    packed = (pix & (SLAB - 1)) | lax.shift_left(
                        j * L + iota, SLAB_SHIFT)
                    plsc.store_compressed(lbuf.at[pl.ds(k * REG + cnt, L)],
                                          packed, mask=mine)
                    new.append(cnt + _pcnt(mine))
                return tuple(new)

            return lax.fori_loop(0, CH // L // NREG, step,
                                 tuple(jnp.int32(0) for _ in range(NREG)))

        def make_ph_b1(bufs, src_ref, base, cnt):
            def ph_b(k, _):
                pk = src_ref[pl.ds(base + k * L, L)]
                locv = pk & (SLAB - 1)
                posv = lax.shift_right_logical(pk, SLAB_SHIFT) & (CH - 1)
                zv = plsc.load_gather(bufs[2], [posv])
                valid = (k * L + iota) < cnt
                cur = plsc.load_gather(depth_ref, [locv])
                w0 = valid & (zv < cur)

                def body(m):
                    plsc.store_scatter(depth_ref, [locv], zv, mask=m)
                    cur2 = plsc.load_gather(depth_ref, [locv])
                    return m & (zv < cur2)

                lax.while_loop(_any, body, w0)
                return 0

            return ph_b

        def make_ph_b2(bufs, off, src_ref, base, cnt):
            def ph_b(k, _):
                pk = src_ref[pl.ds(base + k * L, L)]
                locv = pk & (SLAB - 1)
                posv = lax.shift_right_logical(pk, SLAB_SHIFT) & (CH - 1)
                zv = plsc.load_gather(bufs[2], [posv])
                valid = (k * L + iota) < cnt
                curz = plsc.load_gather(depth_ref, [locv])
                m0 = valid & (zv == curz)

                @pl.when(_any(m0))
                def _():
                    idv = off + posv
                    c0v = plsc.load_gather(bufs[3], [posv])
                    c1v = plsc.load_gather(bufs[4], [posv])
                    c2v = plsc.load_gather(bufs[5], [posv])
                    curi = plsc.load_gather(idbuf_ref, [locv])
                    w0 = m0 & (idv < curi)

                    def body(m):
                        plsc.store_scatter(idbuf_ref, [locv], idv, mask=m)
                        plsc.store_scatter(img0_ref, [locv], c0v, mask=m)
                        plsc.store_scatter(img1_ref, [locv], c1v, mask=m)
                        plsc.store_scatter(img2_ref, [locv], c2v, mask=m)
                        ri = plsc.load_gather(idbuf_ref, [locv])
                        g0 = plsc.load_gather(img0_ref, [locv])
                        g1 = plsc.load_gather(img1_ref, [locv])
                        g2 = plsc.load_gather(img2_ref, [locv])
                        torn = (c0v != g0) | (c1v != g1) | (c2v != g2)
                        return m0 & ((idv < ri) | ((idv == ri) & torn))

                    lax.while_loop(_any, body, w0)
                return 0

            return ph_b

        issue(0, sbufs_a, sem_a, 1)
        issue(1, sbufs_b, sem_b, 1)

        def q1_body(q, poff):
            for par, bufs, sem in ((0, sbufs_a, sem_a),
                                   (1, sbufs_b, sem_b)):
                ci = 2 * q + par
                drain(bufs, sem, 1)
                cnts = ph_a(bufs)
                for r in range(NREG):
                    nb_r = (cnts[r] + (L - 1)) // L
                    lax.fori_loop(0, nb_r,
                                  make_ph_b1(bufs, lbuf, r * REG, cnts[r]),
                                  0)
                total = cnts[0] + cnts[1] + cnts[2] + cnts[3]
                fit = (poff + total) <= (CAP - 16)

                @pl.when(fit)
                def _():
                    dst = poff
                    for r in range(NREG):
                        def cp(i, _, r=r, dst=dst):
                            persist_ref[pl.ds(dst + i * L, L)] = \
                                lbuf[pl.ds(r * REG + i * L, L)]
                            return 0

                        lax.fori_loop(0, (cnts[r] + (L - 1)) // L, cp, 0)
                        dst = dst + cnts[r]
                    segs_ref[ci] = poff
                    lens_ref[ci] = total

                @pl.when(jnp.logical_not(fit))
                def _():
                    segs_ref[ci] = jnp.int32(-1)
                    lens_ref[ci] = jnp.int32(0)

                poff = jnp.where(fit, poff + total, poff)

                @pl.when(ci + 2 < NCHUNK)
                def _():
                    issue(ci + 2, bufs, sem, 1)
            return poff

        lax.fori_loop(0, NCHUNK // 2, q1_body, jnp.int32(0))

        issue(0, sbufs_a, sem_a, 2)
        issue(1, sbufs_b, sem_b, 2)

        def q2_body(q, _):
            for par, bufs, sem in ((0, sbufs_a, sem_a),
                                   (1, sbufs_b, sem_b)):
                ci = 2 * q + par
                off = ci * CH
                drain(bufs, sem, 2)
                s0 = segs_ref[ci]
                slen = lens_ref[ci]

                @pl.when(s0 >= 0)
                def _():
                    nb = (slen + (L - 1)) // L
                    lax.fori_loop(0, nb,
                                  make_ph_b2(bufs, off, persist_ref, s0,
                                             slen), 0)

                @pl.when(s0 < 0)
                def _():
                    cnts = ph_a(bufs)
                    for r in range(NREG):
                        nb_r = (cnts[r] + (L - 1)) // L
                        lax.fori_loop(0, nb_r,
                                      make_ph_b2(bufs, off, lbuf, r * REG,
                                                 cnts[r]), 0)

                @pl.when(ci + 2 < NCHUNK)
                def _():
                    issue(ci + 2, bufs, sem, 2)
            return 0

        lax.fori_loop(0, NCHUNK // 2, q2_body, 0)

        def fin_body(i, _):
            dsl = pl.ds(i * L, L)
            idv = idbuf_ref[dsl]
            dv = depth_ref[dsl]
            valid = idv < NPT
            depth_ref[dsl] = jnp.where(valid, dv, jnp.float32(0.0))
            idbuf_ref[dsl] = jnp.where(valid, idv, jnp.int32(-1))
            return 0

        lax.fori_loop(0, SLAB // L, fin_body, 0)
        gbase = b * HW + o * SLAB
        pltpu.sync_copy(depth_ref, depth_hbm.at[pl.ds(gbase, SLAB)])
        pltpu.sync_copy(idbuf_ref, index_hbm.at[pl.ds(gbase, SLAB)])
        for ch, cref in ((0, img0_ref), (1, img1_ref), (2, img2_ref)):
            ibase = (b * 3 + ch) * HW + o * SLAB
            pltpu.sync_copy(cref, img_hbm.at[pl.ds(ibase, SLAB)])
        return 0

    lax.fori_loop(0, 4, batch_body, 0)


_proj = pl.kernel(
    _body,
    out_type=(
        jax.ShapeDtypeStruct((NB * HW,), jnp.float32),
        jax.ShapeDtypeStruct((NB * 3 * HW,), jnp.float32),
        jax.ShapeDtypeStruct((NB * HW,), jnp.int32),
    ),
    mesh=plsc.VectorSubcoreMesh(core_axis_name="c", subcore_axis_name="s"),
    scratch_types=[
        pltpu.VMEM((SLAB,), jnp.float32),
        pltpu.VMEM((SLAB,), jnp.int32),
        pltpu.VMEM((SLAB,), jnp.float32),
        pltpu.VMEM((SLAB,), jnp.float32),
        pltpu.VMEM((SLAB,), jnp.float32),
        [pltpu.VMEM((CH,), jnp.float32) for _ in range(6)],
        [pltpu.VMEM((CH,), jnp.float32) for _ in range(6)],
        pltpu.VMEM((CH,), jnp.int32),
        pltpu.VMEM((CAP,), jnp.int32),
        pltpu.SMEM((NCHUNK,), jnp.int32),
        pltpu.SMEM((NCHUNK,), jnp.int32),
        pltpu.SemaphoreType.DMA,
        pltpu.SemaphoreType.DMA,
    ],
    compiler_params=pltpu.CompilerParams(needs_layout_passes=False),
)


def kernel(points, colors):
    B, _, N = points.shape
    depth, img, index = _proj(points.reshape(-1), colors.reshape(-1))
    return (depth.reshape(B, H, W),
            img.reshape(B, 3, H, W),
            index.reshape(B, H, W))
